# baseline (speedup 1.0000x reference)
import functools

import jax
import jax.numpy as jnp
from jax import lax
from jax.experimental import pallas as pl
from jax.experimental.pallas import tpu as pltpu
from jax.experimental.pallas import tpu_sc as plsc

NC = 2
NS = 16
NW = NC * NS
L = 16
B = 16384
D = 64
BPW = B // NW
CHUNK = 128
HPW = 256
NPASS = BPW // HPW
NCHUNK = HPW // CHUNK
GROUPS = HPW // L

_mesh = plsc.VectorSubcoreMesh(core_axis_name="c", subcore_axis_name="s")


@functools.partial(
    pl.kernel,
    out_type=jax.ShapeDtypeStruct((B,), jnp.float32),
    mesh=_mesh,
    compiler_params=pltpu.CompilerParams(needs_layout_passes=False),
    scratch_types=[
        pltpu.VMEM((3, BPW // CHUNK, CHUNK), jnp.int32),
        pltpu.VMEM((3, BPW), jnp.int32),
        pltpu.VMEM((HPW, 2 * D), jnp.float32),
        pltpu.VMEM((HPW, 2 * D), jnp.float32),
        pltpu.VMEM((HPW, 2 * D), jnp.float32),
        pltpu.VMEM((BPW,), jnp.float32),
        pltpu.SemaphoreType.DMA,
    ],
)
def _sc_dots(xr_hbm, xh_hbm, user_hbm, item_hbm, out_hbm,
             idx_v, off_v, gu, gi, gj, xout, sem):
    wid = lax.axis_index("s") * NC + lax.axis_index("c")
    base = wid * BPW

    pltpu.sync_copy(xr_hbm.at[wid], idx_v)
    pltpu.sync_copy(xh_hbm.at[wid], off_v)

    for p in range(NPASS):
        handles = []
        for c in range(NCHUNK):
            cg = p * NCHUNK + c
            dst = pl.ds(c * CHUNK, CHUNK)
            handles.append(
                pltpu.async_copy(user_hbm.at[idx_v.at[0, cg]], gu.at[dst], sem))
            handles.append(
                pltpu.async_copy(item_hbm.at[idx_v.at[1, cg]], gi.at[dst], sem))
            handles.append(
                pltpu.async_copy(item_hbm.at[idx_v.at[2, cg]], gj.at[dst], sem))
        for h in handles:
            h.wait()

        def group_body(g, carry):
            rows = g * L + lax.iota(jnp.int32, L)
            grow = p * HPW + g * L
            hu = off_v[0, pl.ds(grow, L)]
            hi = off_v[1, pl.ds(grow, L)]
            hj = off_v[2, pl.ds(grow, L)]
            acc = jnp.zeros((L,), jnp.float32)
            for f in range(D):
                cu = plsc.load_gather(gu, [rows, hu + f])
                ci = plsc.load_gather(gi, [rows, hi + f])
                cj = plsc.load_gather(gj, [rows, hj + f])
                acc = acc + cu * (ci - cj)
            xout[pl.ds(grow, L)] = acc
            return carry

        lax.fori_loop(0, GROUPS, group_body, 0)

    pltpu.sync_copy(xout, out_hbm.at[pl.ds(base, BPW)])


def _loss_body(x_ref, o_ref):
    x = x_ref[...]
    sp = jnp.maximum(-x, 0.0) + jnp.log(1.0 + jnp.exp(-jnp.abs(x)))
    o_ref[...] = jnp.sum(sp, keepdims=True)


def kernel(x, user_embeddings, item_embeddings):
    x = x.astype(jnp.int32)
    xr = (x >> 1).reshape(3, NW, BPW // CHUNK, CHUNK).transpose(1, 0, 2, 3)
    xh = ((x & 1) * D).reshape(3, NW, BPW).transpose(1, 0, 2)
    ue2 = user_embeddings.reshape(user_embeddings.shape[0] // 2, 2 * D)
    ie2 = item_embeddings.reshape(item_embeddings.shape[0] // 2, 2 * D)
    x_uij = _sc_dots(xr, xh, ue2, ie2)
    loss = pl.pallas_call(
        _loss_body,
        out_shape=jax.ShapeDtypeStruct((1, 1), jnp.float32),
    )(x_uij.reshape(B // 128, 128))
    return loss[0, 0]


# speedup vs baseline: 1.0018x; 1.0018x over previous
import functools

import jax
import jax.numpy as jnp
from jax import lax
from jax.experimental import pallas as pl
from jax.experimental.pallas import tpu as pltpu
from jax.experimental.pallas import tpu_sc as plsc

NC = 2
NS = 16
NW = NC * NS
B = 16384
D = 64
BPW = B // NW
CHUNK = 128
NCHUNK = BPW // CHUNK
HPW = 256
NPASS = BPW // HPW
CPP = HPW // CHUNK

_mesh = plsc.VectorSubcoreMesh(core_axis_name="c", subcore_axis_name="s")


@functools.partial(
    pl.kernel,
    out_type=[
        jax.ShapeDtypeStruct((B, 128), jnp.float32),
        jax.ShapeDtypeStruct((B, 128), jnp.float32),
        jax.ShapeDtypeStruct((B, 128), jnp.float32),
    ],
    mesh=_mesh,
    compiler_params=pltpu.CompilerParams(needs_layout_passes=False),
    scratch_types=[
        pltpu.VMEM((3, NCHUNK, CHUNK), jnp.int32),
        pltpu.VMEM((3, HPW, 128), jnp.float32),
        pltpu.SemaphoreType.DMA,
    ],
)
def _sc_gather(xr_hbm, user_hbm, item_hbm, out_u, out_i, out_j,
               idx_v, buf, sem):
    wid = lax.axis_index("s") * NC + lax.axis_index("c")
    base = wid * BPW

    pltpu.sync_copy(xr_hbm.at[wid], idx_v)

    tabs = (user_hbm, item_hbm, item_hbm)
    outs = (out_u, out_i, out_j)
    for p in range(NPASS):
        handles = []
        for t in range(3):
            for c in range(CPP):
                handles.append(pltpu.async_copy(
                    tabs[t].at[idx_v.at[t, p * CPP + c]],
                    buf.at[t, pl.ds(c * CHUNK, CHUNK)], sem))
        for h in handles:
            h.wait()
        handles = []
        for t in range(3):
            handles.append(pltpu.async_copy(
                buf.at[t], outs[t].at[pl.ds(base + p * HPW, HPW)], sem))
        for h in handles:
            h.wait()


BS = 2048
GRID = B // BS


def _loss_body(h_ref, u_ref, i_ref, j_ref, o_ref):
    hu = h_ref[:, 0:1]
    hi = h_ref[:, 1:2]
    hj = h_ref[:, 2:3]
    u = jnp.where(hu > 0, u_ref[:, D:], u_ref[:, :D])
    i = jnp.where(hi > 0, i_ref[:, D:], i_ref[:, :D])
    j = jnp.where(hj > 0, j_ref[:, D:], j_ref[:, :D])
    x = jnp.sum(u * (i - j), axis=1, keepdims=True)
    sp = jnp.maximum(-x, 0.0) + jnp.log1p(jnp.exp(-jnp.abs(x)))
    s = jnp.sum(sp, axis=0, keepdims=True)

    @pl.when(pl.program_id(0) == 0)
    def _init():
        o_ref[...] = jnp.zeros_like(o_ref)

    o_ref[...] += s


def kernel(x, user_embeddings, item_embeddings):
    x = x.astype(jnp.int32)
    xr = (x >> 1).reshape(3, NW, NCHUNK, CHUNK).transpose(1, 0, 2, 3)
    half = (x & 1).astype(jnp.float32).T
    ue2 = user_embeddings.reshape(user_embeddings.shape[0] // 2, 2 * D)
    ie2 = item_embeddings.reshape(item_embeddings.shape[0] // 2, 2 * D)
    gu, gi, gj = _sc_gather(xr, ue2, ie2)
    loss = pl.pallas_call(
        _loss_body,
        grid=(GRID,),
        in_specs=[
            pl.BlockSpec((BS, 3), lambda b: (b, 0)),
            pl.BlockSpec((BS, 128), lambda b: (b, 0)),
            pl.BlockSpec((BS, 128), lambda b: (b, 0)),
            pl.BlockSpec((BS, 128), lambda b: (b, 0)),
        ],
        out_specs=pl.BlockSpec((1, 1), lambda b: (0, 0)),
        out_shape=jax.ShapeDtypeStruct((1, 1), jnp.float32),
    )(half, gu, gi, gj)
    return loss[0, 0]


# speedup vs baseline: 1.0054x; 1.0036x over previous
import functools

import jax
import jax.numpy as jnp
from jax import lax
from jax.experimental import pallas as pl
from jax.experimental.pallas import tpu as pltpu
from jax.experimental.pallas import tpu_sc as plsc

NC = 2
NS = 16
NW = NC * NS
B = 16384
D = 64
BPW = B // NW
CHUNK = 128
HPW = 256
NPASS = BPW // HPW
CPP = HPW // CHUNK

_mesh = plsc.VectorSubcoreMesh(core_axis_name="c", subcore_axis_name="s")


def _gather_body(nt):
    def body(xr_hbm, tab_hbm, *rest):
        outs = rest[:nt]
        idx_v, buf, sem = rest[nt:]
        wid = lax.axis_index("s") * NC + lax.axis_index("c")
        base = wid * BPW

        pltpu.sync_copy(xr_hbm.at[wid], idx_v)

        for p in range(NPASS):
            handles = []
            for t in range(nt):
                for c in range(CPP):
                    handles.append(pltpu.async_copy(
                        tab_hbm.at[idx_v.at[t, p * CPP + c]],
                        buf.at[t, pl.ds(c * CHUNK, CHUNK)], sem))
            for h in handles:
                h.wait()
            handles = []
            for t in range(nt):
                handles.append(pltpu.async_copy(
                    buf.at[t],
                    outs[t].at[pl.ds(base + p * HPW, HPW)], sem))
            for h in handles:
                h.wait()

    return body


def _make_gather(nt):
    return functools.partial(
        pl.kernel,
        out_type=[jax.ShapeDtypeStruct((B, 2 * D), jnp.float32)] * nt,
        mesh=_mesh,
        compiler_params=pltpu.CompilerParams(needs_layout_passes=False),
        scratch_types=[
            pltpu.VMEM((nt, NPASS * CPP, CHUNK), jnp.int32),
            pltpu.VMEM((nt, HPW, 2 * D), jnp.float32),
            pltpu.SemaphoreType.DMA,
        ],
    )(_gather_body(nt))


_gather_user = _make_gather(1)
_gather_item = _make_gather(2)


BS = 2048
GRID = B // BS


def _loss_body(h_ref, u_ref, i_ref, j_ref, o_ref):
    hu = h_ref[:, 0:1]
    hi = h_ref[:, 1:2]
    hj = h_ref[:, 2:3]
    u = jnp.where(hu > 0, u_ref[:, D:], u_ref[:, :D])
    i = jnp.where(hi > 0, i_ref[:, D:], i_ref[:, :D])
    j = jnp.where(hj > 0, j_ref[:, D:], j_ref[:, :D])
    x = jnp.sum(u * (i - j), axis=1, keepdims=True)
    sp = jnp.maximum(-x, 0.0) + jnp.log1p(jnp.exp(-jnp.abs(x)))
    s = jnp.sum(sp, axis=0, keepdims=True)

    @pl.when(pl.program_id(0) == 0)
    def _init():
        o_ref[...] = jnp.zeros_like(o_ref)

    o_ref[...] += s


def kernel(x, user_embeddings, item_embeddings):
    x = x.astype(jnp.int32)
    xr = (x >> 1).reshape(3, NW, NPASS * CPP, CHUNK).transpose(1, 0, 2, 3)
    half = (x & 1).astype(jnp.float32).T
    ue2 = user_embeddings.reshape(user_embeddings.shape[0] // 2, 2 * D)
    ie2 = item_embeddings.reshape(item_embeddings.shape[0] // 2, 2 * D)
    (gu,) = _gather_user(xr[:, 0:1], ue2)
    gi, gj = _gather_item(xr[:, 1:3], ie2)
    loss = pl.pallas_call(
        _loss_body,
        grid=(GRID,),
        in_specs=[
            pl.BlockSpec((BS, 3), lambda b: (b, 0)),
            pl.BlockSpec((BS, 128), lambda b: (b, 0)),
            pl.BlockSpec((BS, 128), lambda b: (b, 0)),
            pl.BlockSpec((BS, 128), lambda b: (b, 0)),
        ],
        out_specs=pl.BlockSpec((1, 1), lambda b: (0, 0)),
        out_shape=jax.ShapeDtypeStruct((1, 1), jnp.float32),
    )(half, gu, gi, gj)
    return loss[0, 0]
